# baseline (speedup 1.0000x reference)
import jax
import jax.numpy as jnp
from jax import lax
from jax.experimental import pallas as pl
from jax.experimental.pallas import tpu as pltpu
from jax.experimental.pallas import tpu_sc as plsc

N_NODES = 10000
N_PAD = 10240
D = 128
W160 = 160
NRELA = 16
E1 = 256000
E2 = 64000
E_TOTAL = E1 + E2
NC = 2
NS = 16
K = 80
EPT = E_TOTAL // (NC * NS)
NCHUNK = EPT // K
STRIPE = N_PAD // NS


def _prep_nodes_body(x_ref, aT_ref, m_ref, b_ref,
                     p1_ref, qa_ref, p2x_ref, cvd_ref):
    x = x_ref[...]
    p1 = jnp.dot(x, aT_ref[0:D, :], preferred_element_type=jnp.float32)
    p2 = jnp.dot(x, aT_ref[D:2 * D, :], preferred_element_type=jnp.float32)
    p1_ref[...] = p1
    m = m_ref[...]
    qa_ref[...] = jnp.dot(p1, m, preferred_element_type=jnp.float32)
    qb = (jnp.dot(p2, m, preferred_element_type=jnp.float32)
          + b_ref[0:1, :])
    nblk = x.shape[0]
    p2x_ref[...] = jnp.concatenate(
        [p2, qb, jnp.zeros((nblk, W160 - D - 2), jnp.float32)], axis=1)
    cvd_ref[...] = lax.dot_general(
        m, aT_ref[2 * D:2 * D + NRELA, :],
        dimension_numbers=(((0,), (1,)), ((), ())),
        preferred_element_type=jnp.float32)


def _post_body(acc0_ref, acc1_ref, p1_ref, a3T_ref, out_ref):
    acc = acc0_ref[...] + acc1_ref[...]
    z = acc[:, D:D + NRELA]
    rs = acc[:, D + NRELA:D + NRELA + 1]
    num = (acc[:, 0:D] + p1_ref[...] * rs
           + jnp.dot(z, a3T_ref[...], preferred_element_type=jnp.float32))
    den = jnp.where(rs == 0.0, 1e-12, rs)
    h = num / den
    out_ref[...] = jnp.where(h > 0.0, h, jnp.exp(h) - 1.0)


def _lane_bcast(v, j):
    return lax.gather(
        v, jnp.full((16, 1), j, jnp.int32),
        dimension_numbers=lax.GatherDimensionNumbers(
            offset_dims=(), collapsed_slice_dims=(0,), start_index_map=(0,)),
        slice_sizes=(1,),
        mode=lax.GatherScatterMode.PROMISE_IN_BOUNDS)


def _sc_body(edata_hbm, ee1_hbm, ee2_hbm, qa_hbm, p2x_hbm, cvd_hbm,
             acc_out,
             edata_v0, ee_v0, edata_v1, ee_v1,
             srcc_v0, srcc_v1, dstc_v,
             prow_v, qa_v, cvd_v,
             acc_sh, semL0, semL1, semG, semS):
    cid = lax.axis_index("c")
    sid = lax.axis_index("s")
    ebase = (cid * NS + sid) * EPT
    nb = sid * STRIPE

    iota16 = jnp.arange(16, dtype=jnp.int32)
    zeros16 = jnp.zeros((16,), jnp.int32)
    zero16f = jnp.zeros((16,), jnp.float32)
    onehot0 = jnp.where(iota16 == 0, 1.0, 0.0).astype(jnp.float32)

    pltpu.sync_copy(cvd_hbm, cvd_v)

    def _zrow(i, carry):
        for r in range(W160 // 16):
            prow_v[i, pl.ds(16 * r, 16)] = zero16f
        return carry

    lax.fori_loop(0, K, _zrow, 0)
    for t in range(STRIPE // K):
        pltpu.sync_copy(prow_v, acc_sh.at[pl.ds(nb + K * t, K)])
    plsc.subcore_barrier()

    cv3 = cvd_v[0, :]
    dv3 = cvd_v[1, :]
    cvb = [_lane_bcast(cv3, c) for c in range(NRELA)]
    dvb = [_lane_bcast(dv3, c) for c in range(NRELA)]

    inv_e = -1.0 / float(E_TOTAL)

    def lin_issue(cbase, edv, ev, sem):
        pltpu.async_copy(edata_hbm.at[pl.ds(cbase, K)], edv, sem)

        @pl.when(cbase < E1)
        def _():
            pltpu.async_copy(ee1_hbm.at[pl.ds(cbase, K)], ev, sem)

        @pl.when(cbase >= E1)
        def _():
            pltpu.async_copy(ee2_hbm.at[pl.ds(cbase - E1, K)], ev, sem)

    def lin_wait(edv, ev, sem):
        pltpu.make_async_copy(edata_hbm.at[pl.ds(0, K)], edv, sem).wait()
        pltpu.make_async_copy(ee1_hbm.at[pl.ds(0, K)], ev, sem).wait()

    def extract(edv, sc):
        for g in range(K // 16):
            rows = 16 * g + iota16
            sl = pl.ds(16 * g, 16)
            sc[sl] = plsc.load_gather(edv, [rows, zeros16])
            dstc_v[sl] = plsc.load_gather(edv, [rows, zeros16 + 1])

    def ind_issue(sc):
        pltpu.async_copy(qa_hbm.at[sc], qa_v, semG)
        pltpu.async_copy(p2x_hbm.at[dstc_v], prow_v, semG)

    def ind_wait(sc):
        pltpu.make_async_copy(qa_hbm.at[sc], qa_v, semG).wait()
        pltpu.make_async_copy(p2x_hbm.at[dstc_v], prow_v, semG).wait()

    def scat_issue(sc):
        pltpu.async_copy(prow_v, acc_sh.at[sc], semS, add=True)

    def scat_wait(sc):
        pltpu.make_async_copy(prow_v, acc_sh.at[sc], semS).wait()

    def _compute(ev):
        for g in range(K // 16):
            rows = 16 * g + iota16
            q1g = plsc.load_gather(qa_v, [rows, zeros16])
            r1g = plsc.load_gather(qa_v, [rows, zeros16 + 1])
            q2g = plsc.load_gather(prow_v, [rows, zeros16 + D])
            r2g = plsc.load_gather(prow_v, [rows, zeros16 + D + 1])
            col = plsc.load_gather(ev, [rows, zeros16])
            qe = col * cvb[0]
            re = col * dvb[0]
            for c in range(1, NRELA):
                col = plsc.load_gather(ev, [rows, zeros16 + c])
                qe = qe + col * cvb[c]
                re = re + col * dvb[c]
            s1 = q1g + q2g + qe
            s2 = r1g + r2g + re
            u = jnp.exp(-2.0 * jnp.abs(s2))
            th = jnp.sign(s2) * (1.0 - u) / (1.0 + u)
            lr = jnp.where(s1 > 0.0, s1, 0.2 * s1)
            ew = jnp.exp(lr * th * inv_e)
            for j in range(16):
                cs = _lane_bcast(ew, j)
                row = 16 * g + j
                for r in range(D // 16):
                    prow_v[row, pl.ds(16 * r, 16)] = (
                        prow_v[row, pl.ds(16 * r, 16)] * cs)
                prow_v[row, pl.ds(D, 16)] = ev[row, :] * cs
                prow_v[row, pl.ds(D + 16, 16)] = cs * onehot0

    def _pair(k, carry):
        b0 = ebase + 2 * k * K
        lin_wait(edata_v0, ee_v0, semL0)
        extract(edata_v0, srcc_v0)
        ind_issue(srcc_v0)
        lin_issue(b0 + K, edata_v1, ee_v1, semL1)
        ind_wait(srcc_v0)
        _compute(ee_v0)
        scat_issue(srcc_v0)
        lin_wait(edata_v1, ee_v1, semL1)
        scat_wait(srcc_v0)
        extract(edata_v1, srcc_v1)
        ind_issue(srcc_v1)
        lin_issue(b0 + 2 * K, edata_v0, ee_v0, semL0)
        ind_wait(srcc_v1)
        _compute(ee_v1)
        scat_issue(srcc_v1)
        scat_wait(srcc_v1)
        return carry

    lin_issue(ebase, edata_v0, ee_v0, semL0)
    lax.fori_loop(0, (NCHUNK - 1) // 2, _pair, 0)
    lin_wait(edata_v0, ee_v0, semL0)
    extract(edata_v0, srcc_v0)
    ind_issue(srcc_v0)
    ind_wait(srcc_v0)
    _compute(ee_v0)
    scat_issue(srcc_v0)
    scat_wait(srcc_v0)
    plsc.subcore_barrier()

    ob = cid * N_PAD + nb
    pltpu.sync_copy(acc_sh.at[pl.ds(nb, STRIPE)], acc_out.at[pl.ds(ob, STRIPE)])


def _sc_call(edata, ee1, ee2, qa, p2x, cvd):
    f = pl.kernel(
        _sc_body,
        out_type=jax.ShapeDtypeStruct((NC * N_PAD, W160), jnp.float32),
        mesh=plsc.VectorSubcoreMesh(core_axis_name="c", subcore_axis_name="s"),
        compiler_params=pltpu.CompilerParams(
            needs_layout_passes=False, use_tc_tiling_on_sc=False),
        scratch_types=[
            pltpu.VMEM((K, 2), jnp.int32),
            pltpu.VMEM((K, NRELA), jnp.float32),
            pltpu.VMEM((K, 2), jnp.int32),
            pltpu.VMEM((K, NRELA), jnp.float32),
            pltpu.VMEM((K,), jnp.int32),
            pltpu.VMEM((K,), jnp.int32),
            pltpu.VMEM((K,), jnp.int32),
            pltpu.VMEM((K, W160), jnp.float32),
            pltpu.VMEM((K, 2), jnp.float32),
            pltpu.VMEM((2, NRELA), jnp.float32),
            pltpu.VMEM_SHARED((N_PAD, W160), jnp.float32),
            pltpu.SemaphoreType.DMA,
            pltpu.SemaphoreType.DMA,
            pltpu.SemaphoreType.DMA,
            pltpu.SemaphoreType.DMA,
        ],
    )
    return f(edata, ee1, ee2, qa, p2x, cvd)


def kernel(input, edge, edge_embed, edge_list_nhop, edge_embed_nhop,
           a, a__2, a_2, W_mlp, b_mlp):
    x = jnp.pad(input, ((0, N_PAD - N_NODES), (0, 0)))
    aT = a.T
    m2 = jnp.stack([a_2[0], W_mlp[0]], axis=1)
    b8 = jnp.pad(jnp.stack([jnp.zeros_like(b_mlp), b_mlp], axis=1),
                 ((0, 7), (0, 0)))
    a3T = aT[2 * D:2 * D + NRELA, :]

    edata = jnp.stack(
        [jnp.concatenate([edge[0], edge_list_nhop[0]]),
         jnp.concatenate([edge[1], edge_list_nhop[1]])], axis=1)

    nblk = 512
    p1, qa, p2x, cvd = pl.pallas_call(
        _prep_nodes_body,
        grid=(N_PAD // nblk,),
        in_specs=[
            pl.BlockSpec((nblk, D), lambda i: (i, 0)),
            pl.BlockSpec((2 * D + NRELA, D), lambda i: (0, 0)),
            pl.BlockSpec((D, 2), lambda i: (0, 0)),
            pl.BlockSpec((8, 2), lambda i: (0, 0)),
        ],
        out_specs=[
            pl.BlockSpec((nblk, D), lambda i: (i, 0)),
            pl.BlockSpec((nblk, 2), lambda i: (i, 0)),
            pl.BlockSpec((nblk, W160), lambda i: (i, 0)),
            pl.BlockSpec((2, NRELA), lambda i: (0, 0)),
        ],
        out_shape=[
            jax.ShapeDtypeStruct((N_PAD, D), jnp.float32),
            jax.ShapeDtypeStruct((N_PAD, 2), jnp.float32),
            jax.ShapeDtypeStruct((N_PAD, W160), jnp.float32),
            jax.ShapeDtypeStruct((2, NRELA), jnp.float32),
        ],
    )(x, aT, m2, b8)

    acc = _sc_call(edata, edge_embed, edge_embed_nhop, qa, p2x, cvd)

    pblk = 512
    h = pl.pallas_call(
        _post_body,
        grid=(N_PAD // pblk,),
        in_specs=[
            pl.BlockSpec((pblk, W160), lambda i: (i, 0)),
            pl.BlockSpec((pblk, W160), lambda i: (i, 0)),
            pl.BlockSpec((pblk, D), lambda i: (i, 0)),
            pl.BlockSpec((NRELA, D), lambda i: (0, 0)),
        ],
        out_specs=pl.BlockSpec((pblk, D), lambda i: (i, 0)),
        out_shape=jax.ShapeDtypeStruct((N_PAD, D), jnp.float32),
    )(acc[:N_PAD], acc[N_PAD:], p1, a3T)

    return h[:N_NODES]


# speedup vs baseline: 1.0191x; 1.0191x over previous
import jax
import jax.numpy as jnp
from jax import lax
from jax.experimental import pallas as pl
from jax.experimental.pallas import tpu as pltpu
from jax.experimental.pallas import tpu_sc as plsc

N_NODES = 10000
N_PAD = 10240
D = 128
NRELA = 16
E1 = 256000
E2 = 64000
E_TOTAL = E1 + E2
NC = 2
NS = 16
K = 80
EPT = E_TOTAL // (NC * NS)
NCHUNK = EPT // K
STRIPE = N_PAD // NS


def _prep_nodes_body(x_ref, aT_ref, m_ref, b_ref,
                     p1_ref, p2_ref, qa_ref, qb_ref, cvd_ref):
    x = x_ref[...]
    p1 = jnp.dot(x, aT_ref[0:D, :], preferred_element_type=jnp.float32)
    p2 = jnp.dot(x, aT_ref[D:2 * D, :], preferred_element_type=jnp.float32)
    p1_ref[...] = p1
    p2_ref[...] = p2
    m = m_ref[...]
    qa_ref[...] = jnp.dot(p1, m, preferred_element_type=jnp.float32)
    qb_ref[...] = (jnp.dot(p2, m, preferred_element_type=jnp.float32)
                   + b_ref[0:1, :])
    cvd_ref[...] = lax.dot_general(
        m, aT_ref[2 * D:2 * D + NRELA, :],
        dimension_numbers=(((0,), (1,)), ((), ())),
        preferred_element_type=jnp.float32)


def _post_body(zr0_ref, zr1_ref, h20_ref, h21_ref, p1_ref, a3T_ref, out_ref):
    zr = zr0_ref[...] + zr1_ref[...]
    z = zr[:, 0:NRELA]
    rs = zr[:, NRELA:NRELA + 1]
    num = (h20_ref[...] + h21_ref[...] + p1_ref[...] * rs
           + jnp.dot(z, a3T_ref[...], preferred_element_type=jnp.float32))
    den = jnp.where(rs == 0.0, 1e-12, rs)
    h = num / den
    out_ref[...] = jnp.where(h > 0.0, h, jnp.exp(h) - 1.0)


def _lane_bcast(v, j):
    return lax.gather(
        v, jnp.full((16, 1), j, jnp.int32),
        dimension_numbers=lax.GatherDimensionNumbers(
            offset_dims=(), collapsed_slice_dims=(0,), start_index_map=(0,)),
        slice_sizes=(1,),
        mode=lax.GatherScatterMode.PROMISE_IN_BOUNDS)


def _sc_body(edata_hbm, ee1_hbm, ee2_hbm, qa_hbm, qb_hbm, p2_hbm, cvd_hbm,
             zr_out, h2_out,
             edata_v0, ee_v0, edata_v1, ee_v1,
             srcc_v0, srcc_v1, dstc_v,
             pay_v, prow_v, qa_v, qb_v, cvd_v,
             zr_sh, h2_sh, semL0, semL1, semG, semS):
    cid = lax.axis_index("c")
    sid = lax.axis_index("s")
    ebase = (cid * NS + sid) * EPT
    nb = sid * STRIPE

    iota16 = jnp.arange(16, dtype=jnp.int32)
    zeros16 = jnp.zeros((16,), jnp.int32)
    zero16f = jnp.zeros((16,), jnp.float32)
    onehot0 = jnp.where(iota16 == 0, 1.0, 0.0).astype(jnp.float32)
    iota16x16 = iota16 * NRELA

    pltpu.sync_copy(cvd_hbm, cvd_v)

    def _zrow(i, carry):
        for r in range(D // 16):
            prow_v[i, pl.ds(16 * r, 16)] = zero16f
        pay_v[i, pl.ds(0, 16)] = zero16f
        pay_v[i, pl.ds(16, 16)] = zero16f
        return carry

    lax.fori_loop(0, K, _zrow, 0)
    for t in range(STRIPE // K):
        pltpu.sync_copy(prow_v, h2_sh.at[pl.ds(nb + K * t, K)])
        pltpu.sync_copy(pay_v, zr_sh.at[pl.ds(nb + K * t, K)])
    plsc.subcore_barrier()

    cv3 = cvd_v[0, :]
    dv3 = cvd_v[1, :]
    cvb = [_lane_bcast(cv3, c) for c in range(NRELA)]
    dvb = [_lane_bcast(dv3, c) for c in range(NRELA)]

    inv_e = -1.0 / float(E_TOTAL)

    def lin_issue(cbase, edv, ev, sem):
        pltpu.async_copy(edata_hbm.at[pl.ds(cbase, K)], edv, sem)

        @pl.when(cbase < E1)
        def _():
            pltpu.async_copy(
                ee1_hbm.at[pl.ds(cbase * NRELA, K * NRELA)], ev, sem)

        @pl.when(cbase >= E1)
        def _():
            pltpu.async_copy(
                ee2_hbm.at[pl.ds((cbase - E1) * NRELA, K * NRELA)], ev, sem)

    def lin_wait(edv, ev, sem):
        pltpu.make_async_copy(edata_hbm.at[pl.ds(0, K)], edv, sem).wait()
        pltpu.make_async_copy(
            ee1_hbm.at[pl.ds(0, K * NRELA)], ev, sem).wait()

    def extract(edv, sc):
        for g in range(K // 16):
            rows = 16 * g + iota16
            sl = pl.ds(16 * g, 16)
            sc[sl] = plsc.load_gather(edv, [rows, zeros16])
            dstc_v[sl] = plsc.load_gather(edv, [rows, zeros16 + 1])

    def ind_issue(sc):
        pltpu.async_copy(qa_hbm.at[sc], qa_v, semG)
        pltpu.async_copy(qb_hbm.at[dstc_v], qb_v, semG)
        pltpu.async_copy(p2_hbm.at[dstc_v], prow_v, semG)

    def ind_wait(sc):
        pltpu.make_async_copy(qa_hbm.at[sc], qa_v, semG).wait()
        pltpu.make_async_copy(qb_hbm.at[dstc_v], qb_v, semG).wait()
        pltpu.make_async_copy(p2_hbm.at[dstc_v], prow_v, semG).wait()

    def scat_issue(sc):
        pltpu.async_copy(pay_v, zr_sh.at[sc], semS, add=True)
        pltpu.async_copy(prow_v, h2_sh.at[sc], semS, add=True)

    def scat_wait(sc):
        pltpu.make_async_copy(pay_v, zr_sh.at[sc], semS).wait()
        pltpu.make_async_copy(prow_v, h2_sh.at[sc], semS).wait()

    def _compute(ev):
        for g in range(K // 16):
            rows = 16 * g + iota16
            q1g = plsc.load_gather(qa_v, [rows, zeros16])
            r1g = plsc.load_gather(qa_v, [rows, zeros16 + 1])
            q2g = plsc.load_gather(qb_v, [rows, zeros16])
            r2g = plsc.load_gather(qb_v, [rows, zeros16 + 1])
            fbase = iota16x16 + (16 * NRELA) * g
            col = plsc.load_gather(ev, [fbase])
            qe = col * cvb[0]
            re = col * dvb[0]
            for c in range(1, NRELA):
                col = plsc.load_gather(ev, [fbase + c])
                qe = qe + col * cvb[c]
                re = re + col * dvb[c]
            s1 = q1g + q2g + qe
            s2 = r1g + r2g + re
            u = jnp.exp(-2.0 * jnp.abs(s2))
            th = jnp.sign(s2) * (1.0 - u) / (1.0 + u)
            lr = jnp.where(s1 > 0.0, s1, 0.2 * s1)
            ew = jnp.exp(lr * th * inv_e)
            for j in range(16):
                cs = _lane_bcast(ew, j)
                row = 16 * g + j
                for r in range(D // 16):
                    prow_v[row, pl.ds(16 * r, 16)] = (
                        prow_v[row, pl.ds(16 * r, 16)] * cs)
                pay_v[row, pl.ds(0, 16)] = ev[pl.ds(row * NRELA, NRELA)] * cs
                pay_v[row, pl.ds(16, 16)] = cs * onehot0

    def _pair(k, carry):
        b0 = ebase + 2 * k * K
        lin_wait(edata_v0, ee_v0, semL0)
        extract(edata_v0, srcc_v0)
        ind_issue(srcc_v0)
        lin_issue(b0 + K, edata_v1, ee_v1, semL1)
        ind_wait(srcc_v0)
        _compute(ee_v0)
        scat_issue(srcc_v0)
        lin_wait(edata_v1, ee_v1, semL1)
        scat_wait(srcc_v0)
        extract(edata_v1, srcc_v1)
        ind_issue(srcc_v1)
        lin_issue(b0 + 2 * K, edata_v0, ee_v0, semL0)
        ind_wait(srcc_v1)
        _compute(ee_v1)
        scat_issue(srcc_v1)
        scat_wait(srcc_v1)
        return carry

    lin_issue(ebase, edata_v0, ee_v0, semL0)
    lax.fori_loop(0, (NCHUNK - 1) // 2, _pair, 0)
    lin_wait(edata_v0, ee_v0, semL0)
    extract(edata_v0, srcc_v0)
    ind_issue(srcc_v0)
    ind_wait(srcc_v0)
    _compute(ee_v0)
    scat_issue(srcc_v0)
    scat_wait(srcc_v0)
    plsc.subcore_barrier()

    ob = cid * N_PAD + nb
    pltpu.sync_copy(zr_sh.at[pl.ds(nb, STRIPE)], zr_out.at[pl.ds(ob, STRIPE)])
    pltpu.sync_copy(h2_sh.at[pl.ds(nb, STRIPE)], h2_out.at[pl.ds(ob, STRIPE)])


def _sc_call(edata, ee1f, ee2f, qa, qb, p2, cvd):
    f = pl.kernel(
        _sc_body,
        out_type=[jax.ShapeDtypeStruct((NC * N_PAD, 32), jnp.float32),
                  jax.ShapeDtypeStruct((NC * N_PAD, D), jnp.float32)],
        mesh=plsc.VectorSubcoreMesh(core_axis_name="c", subcore_axis_name="s"),
        compiler_params=pltpu.CompilerParams(
            needs_layout_passes=False, use_tc_tiling_on_sc=False),
        scratch_types=[
            pltpu.VMEM((K, 2), jnp.int32),
            pltpu.VMEM((K * NRELA,), jnp.float32),
            pltpu.VMEM((K, 2), jnp.int32),
            pltpu.VMEM((K * NRELA,), jnp.float32),
            pltpu.VMEM((K,), jnp.int32),
            pltpu.VMEM((K,), jnp.int32),
            pltpu.VMEM((K,), jnp.int32),
            pltpu.VMEM((K, 32), jnp.float32),
            pltpu.VMEM((K, D), jnp.float32),
            pltpu.VMEM((K, 2), jnp.float32),
            pltpu.VMEM((K, 2), jnp.float32),
            pltpu.VMEM((2, NRELA), jnp.float32),
            pltpu.VMEM_SHARED((N_PAD, 32), jnp.float32),
            pltpu.VMEM_SHARED((N_PAD, D), jnp.float32),
            pltpu.SemaphoreType.DMA,
            pltpu.SemaphoreType.DMA,
            pltpu.SemaphoreType.DMA,
            pltpu.SemaphoreType.DMA,
        ],
    )
    return f(edata, ee1f, ee2f, qa, qb, p2, cvd)


def kernel(input, edge, edge_embed, edge_list_nhop, edge_embed_nhop,
           a, a__2, a_2, W_mlp, b_mlp):
    x = jnp.pad(input, ((0, N_PAD - N_NODES), (0, 0)))
    aT = a.T
    m2 = jnp.stack([a_2[0], W_mlp[0]], axis=1)
    b8 = jnp.pad(jnp.stack([jnp.zeros_like(b_mlp), b_mlp], axis=1),
                 ((0, 7), (0, 0)))
    a3T = aT[2 * D:2 * D + NRELA, :]

    edata = jnp.stack(
        [jnp.concatenate([edge[0], edge_list_nhop[0]]),
         jnp.concatenate([edge[1], edge_list_nhop[1]])], axis=1)
    ee1f = edge_embed.reshape(-1)
    ee2f = edge_embed_nhop.reshape(-1)

    nblk = 512
    p1, p2, qa, qb, cvd = pl.pallas_call(
        _prep_nodes_body,
        grid=(N_PAD // nblk,),
        in_specs=[
            pl.BlockSpec((nblk, D), lambda i: (i, 0)),
            pl.BlockSpec((2 * D + NRELA, D), lambda i: (0, 0)),
            pl.BlockSpec((D, 2), lambda i: (0, 0)),
            pl.BlockSpec((8, 2), lambda i: (0, 0)),
        ],
        out_specs=[
            pl.BlockSpec((nblk, D), lambda i: (i, 0)),
            pl.BlockSpec((nblk, D), lambda i: (i, 0)),
            pl.BlockSpec((nblk, 2), lambda i: (i, 0)),
            pl.BlockSpec((nblk, 2), lambda i: (i, 0)),
            pl.BlockSpec((2, NRELA), lambda i: (0, 0)),
        ],
        out_shape=[
            jax.ShapeDtypeStruct((N_PAD, D), jnp.float32),
            jax.ShapeDtypeStruct((N_PAD, D), jnp.float32),
            jax.ShapeDtypeStruct((N_PAD, 2), jnp.float32),
            jax.ShapeDtypeStruct((N_PAD, 2), jnp.float32),
            jax.ShapeDtypeStruct((2, NRELA), jnp.float32),
        ],
    )(x, aT, m2, b8)

    zr, h2 = _sc_call(edata, ee1f, ee2f, qa, qb, p2, cvd)

    pblk = 512
    h = pl.pallas_call(
        _post_body,
        grid=(N_PAD // pblk,),
        in_specs=[
            pl.BlockSpec((pblk, 32), lambda i: (i, 0)),
            pl.BlockSpec((pblk, 32), lambda i: (i, 0)),
            pl.BlockSpec((pblk, D), lambda i: (i, 0)),
            pl.BlockSpec((pblk, D), lambda i: (i, 0)),
            pl.BlockSpec((pblk, D), lambda i: (i, 0)),
            pl.BlockSpec((NRELA, D), lambda i: (0, 0)),
        ],
        out_specs=pl.BlockSpec((pblk, D), lambda i: (i, 0)),
        out_shape=jax.ShapeDtypeStruct((N_PAD, D), jnp.float32),
    )(zr[:N_PAD], zr[N_PAD:], h2[:N_PAD], h2[N_PAD:], p1, a3T)

    return h[:N_NODES]


# speedup vs baseline: 1.4448x; 1.4178x over previous
import jax
import jax.numpy as jnp
from jax import lax
from jax.experimental import pallas as pl
from jax.experimental.pallas import tpu as pltpu
from jax.experimental.pallas import tpu_sc as plsc

N_NODES = 10000
N_PAD = 10240
D = 128
NRELA = 16
E1 = 256000
E2 = 64000
E_TOTAL = E1 + E2
NC = 2
NS = 16
K = 80
EPT = E_TOTAL // (NC * NS)
NCHUNK = EPT // K
STRIPE = N_PAD // NS


def _prep_nodes_body(x_ref, aT_ref, m_ref, b_ref,
                     p1_ref, p2_ref, qa_ref, qb_ref, cvd_ref):
    x = x_ref[...]
    p1 = jnp.dot(x, aT_ref[0:D, :], preferred_element_type=jnp.float32)
    p2 = jnp.dot(x, aT_ref[D:2 * D, :], preferred_element_type=jnp.float32)
    p1_ref[...] = p1
    p2_ref[...] = p2
    m = m_ref[...]
    qa_ref[...] = jnp.dot(p1, m, preferred_element_type=jnp.float32)
    qb_ref[...] = (jnp.dot(p2, m, preferred_element_type=jnp.float32)
                   + b_ref[0:1, :])
    cvd_ref[...] = lax.dot_general(
        m, aT_ref[2 * D:2 * D + NRELA, :],
        dimension_numbers=(((0,), (1,)), ((), ())),
        preferred_element_type=jnp.float32)


def _post_body(zr0_ref, zr1_ref, h20_ref, h21_ref, p1_ref, a3T_ref, out_ref):
    zr = zr0_ref[...] + zr1_ref[...]
    z = zr[:, 0:NRELA]
    rs = zr[:, NRELA:NRELA + 1]
    num = (h20_ref[...] + h21_ref[...] + p1_ref[...] * rs
           + jnp.dot(z, a3T_ref[...], preferred_element_type=jnp.float32))
    den = jnp.where(rs == 0.0, 1e-12, rs)
    h = num / den
    out_ref[...] = jnp.where(h > 0.0, h, jnp.exp(h) - 1.0)


def _lane_bcast(v, j):
    return lax.gather(
        v, jnp.full((16, 1), j, jnp.int32),
        dimension_numbers=lax.GatherDimensionNumbers(
            offset_dims=(), collapsed_slice_dims=(0,), start_index_map=(0,)),
        slice_sizes=(1,),
        mode=lax.GatherScatterMode.PROMISE_IN_BOUNDS)


def _sc_body(src_hbm, dst_hbm, ee1_hbm, ee2_hbm, qa_hbm, qb_hbm, p2_hbm,
             cvd_hbm,
             zr_out, h2_out,
             src_v0, dst_v0, ee_v0, src_v1, dst_v1, ee_v1,
             pay_v, prow_v, qa_v, qb_v, cvd_v,
             zr_sh, h2_sh, semL0, semL1, semG, semS):
    cid = lax.axis_index("c")
    sid = lax.axis_index("s")
    ebase = (cid * NS + sid) * EPT
    nb = sid * STRIPE

    iota16 = jnp.arange(16, dtype=jnp.int32)
    zeros16 = jnp.zeros((16,), jnp.int32)
    zero16f = jnp.zeros((16,), jnp.float32)
    onehot0 = jnp.where(iota16 == 0, 1.0, 0.0).astype(jnp.float32)
    iota16x16 = iota16 * NRELA

    pltpu.sync_copy(cvd_hbm, cvd_v)

    def _zrow(i, carry):
        for r in range(D // 16):
            prow_v[i, pl.ds(16 * r, 16)] = zero16f
        pay_v[i, pl.ds(0, 16)] = zero16f
        pay_v[i, pl.ds(16, 16)] = zero16f
        return carry

    lax.fori_loop(0, K, _zrow, 0)
    for t in range(STRIPE // K):
        pltpu.sync_copy(prow_v, h2_sh.at[pl.ds(nb + K * t, K)])
        pltpu.sync_copy(pay_v, zr_sh.at[pl.ds(nb + K * t, K)])
    plsc.subcore_barrier()

    cv3 = cvd_v[0, :]
    dv3 = cvd_v[1, :]
    cvb = [_lane_bcast(cv3, c) for c in range(NRELA)]
    dvb = [_lane_bcast(dv3, c) for c in range(NRELA)]

    inv_e = -1.0 / float(E_TOTAL)

    def lin_issue(cbase, sv, dv, ev, sem):
        pltpu.async_copy(src_hbm.at[pl.ds(cbase, K)], sv, sem)
        pltpu.async_copy(dst_hbm.at[pl.ds(cbase, K)], dv, sem)

        @pl.when(cbase < E1)
        def _():
            pltpu.async_copy(
                ee1_hbm.at[pl.ds(cbase * NRELA, K * NRELA)], ev, sem)

        @pl.when(cbase >= E1)
        def _():
            pltpu.async_copy(
                ee2_hbm.at[pl.ds((cbase - E1) * NRELA, K * NRELA)], ev, sem)

    def lin_wait(sv, dv, ev, sem):
        pltpu.make_async_copy(src_hbm.at[pl.ds(0, K)], sv, sem).wait()
        pltpu.make_async_copy(dst_hbm.at[pl.ds(0, K)], dv, sem).wait()
        pltpu.make_async_copy(
            ee1_hbm.at[pl.ds(0, K * NRELA)], ev, sem).wait()

    def ind_issue(sc, dc):
        pltpu.async_copy(qa_hbm.at[sc], qa_v, semG)
        pltpu.async_copy(qb_hbm.at[dc], qb_v, semG)
        pltpu.async_copy(p2_hbm.at[dc], prow_v, semG)

    def ind_wait(sc, dc):
        pltpu.make_async_copy(qa_hbm.at[sc], qa_v, semG).wait()
        pltpu.make_async_copy(qb_hbm.at[dc], qb_v, semG).wait()
        pltpu.make_async_copy(p2_hbm.at[dc], prow_v, semG).wait()

    def scat_issue(sc):
        pltpu.async_copy(pay_v, zr_sh.at[sc], semS, add=True)
        pltpu.async_copy(prow_v, h2_sh.at[sc], semS, add=True)

    def scat_wait(sc):
        pltpu.make_async_copy(pay_v, zr_sh.at[sc], semS).wait()
        pltpu.make_async_copy(prow_v, h2_sh.at[sc], semS).wait()

    def _compute(ev):
        for g in range(K // 16):
            rows = 16 * g + iota16
            q1g = plsc.load_gather(qa_v, [rows, zeros16])
            r1g = plsc.load_gather(qa_v, [rows, zeros16 + 1])
            q2g = plsc.load_gather(qb_v, [rows, zeros16])
            r2g = plsc.load_gather(qb_v, [rows, zeros16 + 1])
            fbase = iota16x16 + (16 * NRELA) * g
            col = plsc.load_gather(ev, [fbase])
            qe = col * cvb[0]
            re = col * dvb[0]
            for c in range(1, NRELA):
                col = plsc.load_gather(ev, [fbase + c])
                qe = qe + col * cvb[c]
                re = re + col * dvb[c]
            s1 = q1g + q2g + qe
            s2 = r1g + r2g + re
            u = jnp.exp(-2.0 * jnp.abs(s2))
            th = jnp.sign(s2) * (1.0 - u) / (1.0 + u)
            lr = jnp.where(s1 > 0.0, s1, 0.2 * s1)
            ew = jnp.exp(lr * th * inv_e)
            for j in range(16):
                cs = _lane_bcast(ew, j)
                row = 16 * g + j
                for r in range(D // 16):
                    prow_v[row, pl.ds(16 * r, 16)] = (
                        prow_v[row, pl.ds(16 * r, 16)] * cs)
                pay_v[row, pl.ds(0, 16)] = ev[pl.ds(row * NRELA, NRELA)] * cs
                pay_v[row, pl.ds(16, 16)] = cs * onehot0

    def _pair(k, carry):
        b0 = ebase + 2 * k * K
        lin_wait(src_v0, dst_v0, ee_v0, semL0)
        ind_issue(src_v0, dst_v0)
        lin_issue(b0 + K, src_v1, dst_v1, ee_v1, semL1)
        ind_wait(src_v0, dst_v0)
        _compute(ee_v0)
        scat_issue(src_v0)
        lin_wait(src_v1, dst_v1, ee_v1, semL1)
        scat_wait(src_v0)
        ind_issue(src_v1, dst_v1)
        lin_issue(b0 + 2 * K, src_v0, dst_v0, ee_v0, semL0)
        ind_wait(src_v1, dst_v1)
        _compute(ee_v1)
        scat_issue(src_v1)
        scat_wait(src_v1)
        return carry

    lin_issue(ebase, src_v0, dst_v0, ee_v0, semL0)
    lax.fori_loop(0, (NCHUNK - 1) // 2, _pair, 0)
    lin_wait(src_v0, dst_v0, ee_v0, semL0)
    ind_issue(src_v0, dst_v0)
    ind_wait(src_v0, dst_v0)
    _compute(ee_v0)
    scat_issue(src_v0)
    scat_wait(src_v0)
    plsc.subcore_barrier()

    ob = cid * N_PAD + nb
    pltpu.sync_copy(zr_sh.at[pl.ds(nb, STRIPE)], zr_out.at[pl.ds(ob, STRIPE)])
    pltpu.sync_copy(h2_sh.at[pl.ds(nb, STRIPE)], h2_out.at[pl.ds(ob, STRIPE)])


def _sc_call(src, dst, ee1f, ee2f, qa, qb, p2, cvd):
    f = pl.kernel(
        _sc_body,
        out_type=[jax.ShapeDtypeStruct((NC * N_PAD, 32), jnp.float32),
                  jax.ShapeDtypeStruct((NC * N_PAD, D), jnp.float32)],
        mesh=plsc.VectorSubcoreMesh(core_axis_name="c", subcore_axis_name="s"),
        compiler_params=pltpu.CompilerParams(
            needs_layout_passes=False, use_tc_tiling_on_sc=False),
        scratch_types=[
            pltpu.VMEM((K,), jnp.int32),
            pltpu.VMEM((K,), jnp.int32),
            pltpu.VMEM((K * NRELA,), jnp.float32),
            pltpu.VMEM((K,), jnp.int32),
            pltpu.VMEM((K,), jnp.int32),
            pltpu.VMEM((K * NRELA,), jnp.float32),
            pltpu.VMEM((K, 32), jnp.float32),
            pltpu.VMEM((K, D), jnp.float32),
            pltpu.VMEM((K, 2), jnp.float32),
            pltpu.VMEM((K, 2), jnp.float32),
            pltpu.VMEM((2, NRELA), jnp.float32),
            pltpu.VMEM_SHARED((N_PAD, 32), jnp.float32),
            pltpu.VMEM_SHARED((N_PAD, D), jnp.float32),
            pltpu.SemaphoreType.DMA,
            pltpu.SemaphoreType.DMA,
            pltpu.SemaphoreType.DMA,
            pltpu.SemaphoreType.DMA,
        ],
    )
    return f(src, dst, ee1f, ee2f, qa, qb, p2, cvd)


def kernel(input, edge, edge_embed, edge_list_nhop, edge_embed_nhop,
           a, a__2, a_2, W_mlp, b_mlp):
    x = jnp.pad(input, ((0, N_PAD - N_NODES), (0, 0)))
    aT = a.T
    m2 = jnp.stack([a_2[0], W_mlp[0]], axis=1)
    b8 = jnp.pad(jnp.stack([jnp.zeros_like(b_mlp), b_mlp], axis=1),
                 ((0, 7), (0, 0)))
    a3T = aT[2 * D:2 * D + NRELA, :]

    src = jnp.concatenate([edge[0], edge_list_nhop[0]])
    dst = jnp.concatenate([edge[1], edge_list_nhop[1]])
    ee1f = edge_embed.reshape(-1)
    ee2f = edge_embed_nhop.reshape(-1)

    nblk = 512
    p1, p2, qa, qb, cvd = pl.pallas_call(
        _prep_nodes_body,
        grid=(N_PAD // nblk,),
        in_specs=[
            pl.BlockSpec((nblk, D), lambda i: (i, 0)),
            pl.BlockSpec((2 * D + NRELA, D), lambda i: (0, 0)),
            pl.BlockSpec((D, 2), lambda i: (0, 0)),
            pl.BlockSpec((8, 2), lambda i: (0, 0)),
        ],
        out_specs=[
            pl.BlockSpec((nblk, D), lambda i: (i, 0)),
            pl.BlockSpec((nblk, D), lambda i: (i, 0)),
            pl.BlockSpec((nblk, 2), lambda i: (i, 0)),
            pl.BlockSpec((nblk, 2), lambda i: (i, 0)),
            pl.BlockSpec((2, NRELA), lambda i: (0, 0)),
        ],
        out_shape=[
            jax.ShapeDtypeStruct((N_PAD, D), jnp.float32),
            jax.ShapeDtypeStruct((N_PAD, D), jnp.float32),
            jax.ShapeDtypeStruct((N_PAD, 2), jnp.float32),
            jax.ShapeDtypeStruct((N_PAD, 2), jnp.float32),
            jax.ShapeDtypeStruct((2, NRELA), jnp.float32),
        ],
    )(x, aT, m2, b8)

    zr, h2 = _sc_call(src, dst, ee1f, ee2f, qa, qb, p2, cvd)

    pblk = 512
    h = pl.pallas_call(
        _post_body,
        grid=(N_PAD // pblk,),
        in_specs=[
            pl.BlockSpec((pblk, 32), lambda i: (i, 0)),
            pl.BlockSpec((pblk, 32), lambda i: (i, 0)),
            pl.BlockSpec((pblk, D), lambda i: (i, 0)),
            pl.BlockSpec((pblk, D), lambda i: (i, 0)),
            pl.BlockSpec((pblk, D), lambda i: (i, 0)),
            pl.BlockSpec((NRELA, D), lambda i: (0, 0)),
        ],
        out_specs=pl.BlockSpec((pblk, D), lambda i: (i, 0)),
        out_shape=jax.ShapeDtypeStruct((N_PAD, D), jnp.float32),
    )(zr[:N_PAD], zr[N_PAD:], h2[:N_PAD], h2[N_PAD:], p1, a3T)

    return h[:N_NODES]
